# speedup vs baseline: 4.4963x; 4.4963x over previous
import functools

import jax
import jax.numpy as jnp
from jax import lax
from jax.experimental import pallas as pl
from jax.experimental.pallas import tpu as pltpu
from jax.experimental.pallas import tpu_sc as plsc

_DIM = 128
_NC = 2
_NS = 16
_NW = _NC * _NS
_R = 128


@functools.lru_cache(maxsize=None)
def _tied_embed_sc(total: int):
    per_w = total // _NW
    n_chunks = per_w // _R
    mesh = plsc.VectorSubcoreMesh(core_axis_name="c", subcore_axis_name="s")

    def body(ids_hbm, w_hbm, b_hbm, out_hbm, idx_v, rows_v, sem):
        wid = lax.axis_index("s") * _NC + lax.axis_index("c")
        base = wid * per_w
        pltpu.sync_copy(ids_hbm.at[wid], idx_v)

        @pl.loop(0, n_chunks)
        def _chunk(j):
            pltpu.async_copy(w_hbm.at[idx_v.at[j]], rows_v, sem).wait()
            pltpu.async_copy(b_hbm.at[idx_v.at[j]], rows_v, sem, add=True).wait()
            pltpu.sync_copy(rows_v, out_hbm.at[pl.ds(base + j * _R, _R)])

    return pl.kernel(
        body,
        out_type=jax.ShapeDtypeStruct((total, _DIM), jnp.float32),
        mesh=mesh,
        scratch_types=[
            pltpu.VMEM((n_chunks, _R), jnp.int32),
            pltpu.VMEM((_R, _DIM), jnp.float32),
            pltpu.SemaphoreType.DMA,
        ],
    )


def kernel(input_ids, weight, bias):
    B, H = input_ids.shape
    total = B * H
    ids = input_ids.reshape(_NW, total // (_NW * _R), _R).astype(jnp.int32)
    out = _tied_embed_sc(total)(ids, weight, bias)
    return out.reshape(B, H, _DIM)


# speedup vs baseline: 5.4216x; 1.2058x over previous
import functools

import jax
import jax.numpy as jnp
from jax import lax
from jax.experimental import pallas as pl
from jax.experimental.pallas import tpu as pltpu
from jax.experimental.pallas import tpu_sc as plsc

_DIM = 128
_NC = 2
_NS = 16
_NW = _NC * _NS
_R = 128


_NBUF = 5


@functools.lru_cache(maxsize=None)
def _tied_embed_sc(total: int):
    per_w = total // _NW
    n_chunks = per_w // _R
    mesh = plsc.VectorSubcoreMesh(core_axis_name="c", subcore_axis_name="s")

    def body(ids_hbm, w_hbm, b_hbm, out_hbm, idx_v, rows_v, sem_g, sem_a, sem_s):
        wid = lax.axis_index("s") * _NC + lax.axis_index("c")
        base = wid * per_w
        pltpu.sync_copy(ids_hbm.at[wid], idx_v)

        @pl.loop(0, n_chunks, step=_NBUF)
        def _group(j0):
            gathers = []
            for b in range(_NBUF):
                gathers.append(
                    pltpu.async_copy(
                        w_hbm.at[idx_v.at[j0 + b]], rows_v.at[b], sem_g.at[b]))
            adds = []
            for b in range(_NBUF):
                gathers[b].wait()
                adds.append(
                    pltpu.async_copy(
                        b_hbm.at[idx_v.at[j0 + b]], rows_v.at[b], sem_a.at[b],
                        add=True))
            stores = []
            for b in range(_NBUF):
                adds[b].wait()
                stores.append(
                    pltpu.async_copy(
                        rows_v.at[b],
                        out_hbm.at[pl.ds(base + (j0 + b) * _R, _R)],
                        sem_s.at[b]))
            for b in range(_NBUF):
                stores[b].wait()

    return pl.kernel(
        body,
        out_type=jax.ShapeDtypeStruct((total, _DIM), jnp.float32),
        mesh=mesh,
        scratch_types=[
            pltpu.VMEM((n_chunks, _R), jnp.int32),
            pltpu.VMEM((_NBUF, _R, _DIM), jnp.float32),
            pltpu.SemaphoreType.DMA((_NBUF,)),
            pltpu.SemaphoreType.DMA((_NBUF,)),
            pltpu.SemaphoreType.DMA((_NBUF,)),
        ],
    )


def kernel(input_ids, weight, bias):
    B, H = input_ids.shape
    total = B * H
    ids = input_ids.reshape(_NW, total // (_NW * _R), _R).astype(jnp.int32)
    out = _tied_embed_sc(total)(ids, weight, bias)
    return out.reshape(B, H, _DIM)
